# Initial kernel scaffold; baseline (speedup 1.0000x reference)
import jax
import jax.numpy as jnp
from jax.experimental import pallas as pl


def kernel(x, edge_index, eps, W1, b1, W2, b2, gamma, beta, Wl, bl):
    raise NotImplementedError("write your pallas kernel here")


# speedup vs baseline: 6.7586x; 6.7586x over previous
import functools

import jax
import jax.numpy as jnp
from jax import lax
from jax.experimental import pallas as pl
from jax.experimental.pallas import tpu as pltpu
from jax.experimental.pallas import tpu_sc as plsc

N_NODES = 10000
N_EDGES = 320000
D_FEAT = 128
HID = 64

NC = 2
NS = 16
NW = NC * NS
EPW = N_EDGES // NW
CHUNK = 128
NCHUNK = EPW // CHUNK
REM = EPW - NCHUNK * CHUNK
RPT = N_NODES // NS


def _proj_body(x_ref, w1_ref, y_ref):
    y_ref[...] = lax.dot_general(
        x_ref[...], w1_ref[...], (((1,), (1,)), ((), ())),
        preferred_element_type=jnp.float32)


def _project(x, w1):
    blk = 1000
    return pl.pallas_call(
        _proj_body,
        grid=(N_NODES // blk,),
        in_specs=[
            pl.BlockSpec((blk, D_FEAT), lambda i: (i, 0)),
            pl.BlockSpec((HID, D_FEAT), lambda i: (0, 0)),
        ],
        out_specs=pl.BlockSpec((blk, HID), lambda i: (i, 0)),
        out_shape=jax.ShapeDtypeStruct((N_NODES, HID), jnp.float32),
    )(x, w1)


def _segsum_body(y_hbm, src_hbm, dst_hbm, out_hbm,
                 si, di, rows, si2, di2, rows2, zbuf, acc, sem):
    c = lax.axis_index("c")
    s = lax.axis_index("s")
    wid = s * NC + c

    z16 = jnp.zeros((16,), jnp.float32)

    def _zero(i, carry):
        zbuf[i // 4, pl.ds((i % 4) * 16, 16)] = z16
        return carry

    lax.fori_loop(0, RPT * 4, _zero, 0)
    pltpu.sync_copy(zbuf, acc.at[pl.ds(s * RPT, RPT)])
    plsc.subcore_barrier()

    base = wid * EPW

    def _edges(i, carry):
        off = base + i * CHUNK
        pltpu.sync_copy(src_hbm.at[pl.ds(off, CHUNK)], si)
        pltpu.sync_copy(dst_hbm.at[pl.ds(off, CHUNK)], di)
        pltpu.async_copy(y_hbm.at[si], rows, sem).wait()
        pltpu.sync_copy(rows, acc.at[di], add=True)
        return carry

    lax.fori_loop(0, NCHUNK, _edges, 0)

    off = base + NCHUNK * CHUNK
    pltpu.sync_copy(src_hbm.at[pl.ds(off, REM)], si2)
    pltpu.sync_copy(dst_hbm.at[pl.ds(off, REM)], di2)
    pltpu.async_copy(y_hbm.at[si2], rows2, sem).wait()
    pltpu.sync_copy(rows2, acc.at[di2], add=True)

    plsc.subcore_barrier()
    pltpu.sync_copy(acc.at[pl.ds(s * RPT, RPT)],
                    out_hbm.at[pl.ds(c * N_NODES + s * RPT, RPT)])


def _segsum(y, src, dst):
    mesh = plsc.VectorSubcoreMesh(core_axis_name="c", subcore_axis_name="s")
    k = functools.partial(
        pl.kernel,
        out_type=jax.ShapeDtypeStruct((NC * N_NODES, HID), jnp.float32),
        mesh=mesh,
        scratch_types=[
            pltpu.VMEM((CHUNK,), jnp.int32),
            pltpu.VMEM((CHUNK,), jnp.int32),
            pltpu.VMEM((CHUNK, HID), jnp.float32),
            pltpu.VMEM((REM,), jnp.int32),
            pltpu.VMEM((REM,), jnp.int32),
            pltpu.VMEM((REM, HID), jnp.float32),
            pltpu.VMEM((RPT, HID), jnp.float32),
            pltpu.VMEM_SHARED((N_NODES, HID), jnp.float32),
            pltpu.SemaphoreType.DMA,
        ],
        compiler_params=pltpu.CompilerParams(use_tc_tiling_on_sc=False),
    )(_segsum_body)
    return k(y, src, dst)


def _finish_body(y_ref, agg_ref, eps_ref, b1_ref, w2_ref, b2_ref,
                 g_ref, be_ref, wl_ref, bl_ref, o_ref):
    agg = agg_ref[0] + agg_ref[1]
    h = (1.0 + eps_ref[0, 0]) * y_ref[...] + agg + b1_ref[...]
    h = jnp.maximum(h, 0.0)
    h = lax.dot_general(h, w2_ref[...], (((1,), (1,)), ((), ())),
                        preferred_element_type=jnp.float32) + b2_ref[...]
    mean = jnp.mean(h, axis=0, keepdims=True)
    var = jnp.mean((h - mean) ** 2, axis=0, keepdims=True)
    h = (h - mean) * lax.rsqrt(var + 1e-5) * g_ref[...] + be_ref[...]
    h = jnp.maximum(h, 0.0)
    o_ref[...] = lax.dot_general(h, wl_ref[...], (((1,), (1,)), ((), ())),
                                 preferred_element_type=jnp.float32) + bl_ref[...]


def _finish(y, aggp, eps, b1, w2, b2, gamma, beta, wl, bl):
    return pl.pallas_call(
        _finish_body,
        out_shape=jax.ShapeDtypeStruct((N_NODES, HID), jnp.float32),
    )(y, aggp, eps, b1, w2, b2, gamma, beta, wl, bl)


def kernel(x, edge_index, eps, W1, b1, W2, b2, gamma, beta, Wl, bl):
    src = edge_index[0].astype(jnp.int32)
    dst = edge_index[1].astype(jnp.int32)
    y = _project(x, W1)
    aggp = _segsum(y, src, dst).reshape(NC, N_NODES, HID)
    return _finish(y, aggp,
                   jnp.asarray(eps, jnp.float32).reshape(1, 1),
                   b1.reshape(1, HID), W2, b2.reshape(1, HID),
                   gamma.reshape(1, HID), beta.reshape(1, HID),
                   Wl, bl.reshape(1, HID))
